# baseline (speedup 1.0000x reference)
import functools

import jax
import jax.numpy as jnp
from jax import lax
from jax.experimental import pallas as pl
from jax.experimental.pallas import tpu as pltpu
from jax.experimental.pallas import tpu_sc as plsc

NUM_FIELDS = 26
BATCH = 16384
EMBED_DIM = 32
VOCAB = 1000000

TBLK = 12800
TGRID = -(-VOCAB // TBLK)
TOUT = TBLK * EMBED_DIM // 128
OUT_ROWS = 250048
VOCAB_PAD = OUT_ROWS * 4


def _table_body(in_ref, out_ref):
  for s in range(TBLK // 512):
    stacked = jnp.concatenate(
        [in_ref[:, 512 * s + 128 * c:512 * s + 128 * (c + 1)]
         for c in range(4)], axis=0)
    out_ref[128 * s:128 * (s + 1), :] = stacked.T


_table_transpose = pl.pallas_call(
    _table_body,
    grid=(TGRID,),
    in_specs=[pl.BlockSpec((EMBED_DIM, TBLK), lambda i: (0, i))],
    out_specs=pl.BlockSpec((TOUT, 128), lambda i: (i, 0)),
    out_shape=jax.ShapeDtypeStruct((OUT_ROWS, 128), jnp.float32),
)

NUM_WORKERS = 32
FIELDS_PAD = 28
B_PER_W = BATCH // NUM_WORKERS
B_CHUNK = 32
NUM_CHUNKS = B_PER_W // B_CHUNK
RAW_CHUNK = B_CHUNK * NUM_FIELDS
PAD_CHUNK = B_CHUNK * FIELDS_PAD
STREAM_LEN = 112
STREAMS_PER_CHUNK = PAD_CHUNK // STREAM_LEN
NBUF = 2
L = 16
TOTAL_PAD = BATCH * FIELDS_PAD


def _make_gather():
  mesh = plsc.VectorSubcoreMesh(core_axis_name="c", subcore_axis_name="s")

  @functools.partial(
      pl.kernel,
      mesh=mesh,
      out_type=jax.ShapeDtypeStruct((TOTAL_PAD // STREAM_LEN, STREAM_LEN,
                                     EMBED_DIM), jnp.float32),
      scratch_types=[
          pltpu.VMEM((NBUF, RAW_CHUNK), jnp.int32),
          pltpu.VMEM((NBUF, PAD_CHUNK), jnp.int32),
          pltpu.VMEM((NBUF, STREAMS_PER_CHUNK, STREAM_LEN, EMBED_DIM),
                     jnp.float32),
          pltpu.SemaphoreType.DMA,
          pltpu.SemaphoreType.DMA,
      ],
      compiler_params=pltpu.CompilerParams(use_tc_tiling_on_sc=False,
                                           needs_layout_passes=False),
  )
  def gather_kernel(table_hbm, x_hbm, out_hbm, raw_v, idx_v, rows_v,
                    sem0, sem1):
    sems = (sem0, sem1)
    wid = lax.axis_index("s") * 2 + lax.axis_index("c")
    base = wid * B_PER_W * NUM_FIELDS
    obase = wid * B_PER_W * FIELDS_PAD // STREAM_LEN

    def fire(g, b):
      pltpu.sync_copy(x_hbm.at[pl.ds(base + g * RAW_CHUNK, RAW_CHUNK)],
                      raw_v.at[b])
      for k in range(PAD_CHUNK // L):
        flat = lax.iota(jnp.int32, L) + jnp.int32(k * L)
        br = (flat * jnp.int32(2341)) >> 13
        br = br >> 3
        f = flat - br * jnp.int32(FIELDS_PAD)
        srcpos = br * jnp.int32(NUM_FIELDS) + jnp.minimum(
            f, jnp.int32(NUM_FIELDS - 1))
        i = plsc.load_gather(raw_v.at[b], [srcpos])
        pi = ((i & jnp.int32(~511)) | ((i & jnp.int32(127)) << 2)
              | ((i >> 7) & jnp.int32(3)))
        idx_v[b, pl.ds(k * L, L)] = jnp.where(
            f < jnp.int32(NUM_FIELDS), pi, jnp.int32(0))
      for j in range(STREAMS_PER_CHUNK):
        pltpu.async_copy(
            table_hbm.at[idx_v.at[b].at[pl.ds(j * STREAM_LEN, STREAM_LEN)]],
            rows_v.at[b].at[j],
            sems[b])

    def drain_and_writeback(g, b):
      pltpu.make_async_copy(
          out_hbm.at[pl.ds(0, STREAMS_PER_CHUNK)], rows_v.at[b],
          sems[b]).wait()
      pltpu.sync_copy(
          rows_v.at[b],
          out_hbm.at[pl.ds(obase + g * STREAMS_PER_CHUNK,
                           STREAMS_PER_CHUNK)])

    fire(0, 0)
    fire(1, 1)

    def body(k, _):
      for b in range(NBUF):
        g = NBUF * k + b
        drain_and_writeback(g, b)

        @pl.when(g + NBUF < NUM_CHUNKS)
        def _():
          fire(g + NBUF, b)
      return ()

    lax.fori_loop(0, NUM_CHUNKS // NBUF, body, (), unroll=False)

  return gather_kernel


_gather = _make_gather()

OBLK_IN = BATCH * FIELDS_PAD // 128 // 128
OGRID = BATCH // 128


def _out_body(in_ref, out_ref):
  in3 = in_ref[...].reshape(128, 7, 128)
  for q in range(7):
    sq = in3[:, q, :].T
    if 128 * (q + 1) <= NUM_FIELDS * EMBED_DIM:
      out_ref[128 * q:128 * (q + 1), :] = sq
    else:
      out_ref[128 * q:NUM_FIELDS * EMBED_DIM, :] = \
          sq[:NUM_FIELDS * EMBED_DIM - 128 * q, :]


_out_transpose = pl.pallas_call(
    _out_body,
    grid=(OGRID,),
    in_specs=[pl.BlockSpec((FIELDS_PAD * 32, 128), lambda i: (i, 0))],
    out_specs=pl.BlockSpec((NUM_FIELDS * EMBED_DIM, 128), lambda i: (0, i)),
    out_shape=jax.ShapeDtypeStruct((NUM_FIELDS * EMBED_DIM, BATCH),
                                   jnp.float32),
)


@jax.jit
def kernel(x, embs):
  table = _table_transpose(embs.T).reshape(VOCAB_PAD, EMBED_DIM)
  out3 = _gather(table, x.reshape(-1))
  outt = _out_transpose(out3.reshape(TOTAL_PAD * EMBED_DIM // 128, 128))
  return outt.T


# speedup vs baseline: 2.3640x; 2.3640x over previous
import functools

import jax
import jax.numpy as jnp
from jax import lax
from jax.experimental import pallas as pl
from jax.experimental.pallas import tpu as pltpu
from jax.experimental.pallas import tpu_sc as plsc

NUM_FIELDS = 26
BATCH = 16384
EMBED_DIM = 32
VOCAB = 1000000

TBLK = 12800
TGRID = -(-VOCAB // TBLK)
TOUT = TBLK * EMBED_DIM // 128
VROWS = TGRID * TBLK
OUT_ROWS = 250048
VOCAB_PAD = OUT_ROWS * 4


def _transpose_body(in_ref, out_ref):
  for s in range(TBLK // 512):
    stacked = jnp.concatenate(
        [in_ref[:, 512 * s + 128 * c:512 * s + 128 * (c + 1)]
         for c in range(4)], axis=0)
    out_ref[128 * s:128 * (s + 1), :] = stacked.T


_transpose = pl.pallas_call(
    _transpose_body,
    grid=(TGRID,),
    in_specs=[pl.BlockSpec((EMBED_DIM, TBLK), lambda i: (0, i))],
    out_specs=pl.BlockSpec((TOUT, 128), lambda i: (i, 0)),
    out_shape=jax.ShapeDtypeStruct((OUT_ROWS, 128), jnp.float32),
)

NUM_WORKERS = 32
TOTAL = BATCH * NUM_FIELDS
PER_WORKER = TOTAL // NUM_WORKERS
STREAM_LEN = 104
STREAMS_PER_CHUNK = 8
CHUNK = STREAM_LEN * STREAMS_PER_CHUNK
NUM_CHUNKS = PER_WORKER // CHUNK
NBUF = 2
L = 16


def _make_gather():
  mesh = plsc.VectorSubcoreMesh(core_axis_name="c", subcore_axis_name="s")

  @functools.partial(
      pl.kernel,
      mesh=mesh,
      out_type=jax.ShapeDtypeStruct((TOTAL // STREAM_LEN, STREAM_LEN,
                                     EMBED_DIM), jnp.float32),
      scratch_types=[
          pltpu.VMEM((NBUF, CHUNK), jnp.int32),
          pltpu.VMEM((NBUF, STREAMS_PER_CHUNK, STREAM_LEN, EMBED_DIM),
                     jnp.float32),
          pltpu.SemaphoreType.DMA,
          pltpu.SemaphoreType.DMA,
      ],
      compiler_params=pltpu.CompilerParams(use_tc_tiling_on_sc=False),
  )
  def gather_kernel(table_hbm, x_hbm, out_hbm, idx_v, rows_v, sem0, sem1):
    sems = (sem0, sem1)
    wid = lax.axis_index("s") * 2 + lax.axis_index("c")
    base = wid * PER_WORKER

    def fire(g, b):
      pltpu.sync_copy(x_hbm.at[pl.ds(base + g * CHUNK, CHUNK)], idx_v.at[b])
      for k in range(CHUNK // L):
        i = idx_v[b, pl.ds(k * L, L)]
        pi = ((i & jnp.int32(~511)) | ((i & jnp.int32(127)) << 2)
              | ((i >> 7) & jnp.int32(3)))
        idx_v[b, pl.ds(k * L, L)] = pi
      for j in range(STREAMS_PER_CHUNK):
        pltpu.async_copy(
            table_hbm.at[idx_v.at[b].at[pl.ds(j * STREAM_LEN, STREAM_LEN)]],
            rows_v.at[b].at[j],
            sems[b])

    def drain_and_writeback(g, b):
      pltpu.make_async_copy(
          out_hbm.at[pl.ds(0, STREAMS_PER_CHUNK)], rows_v.at[b],
          sems[b]).wait()
      pltpu.sync_copy(
          rows_v.at[b],
          out_hbm.at[pl.ds((base + g * CHUNK) // STREAM_LEN,
                           STREAMS_PER_CHUNK)])

    fire(0, 0)
    fire(1, 1)

    def body(k, _):
      for b in range(NBUF):
        g = NBUF * k + b
        drain_and_writeback(g, b)

        @pl.when(g + NBUF < NUM_CHUNKS)
        def _():
          fire(g + NBUF, b)
      return ()

    lax.fori_loop(0, NUM_CHUNKS // NBUF, body, (), unroll=False)

  return gather_kernel


_gather = _make_gather()


@jax.jit
def kernel(x, embs):
  table = _transpose(embs.T).reshape(VOCAB_PAD, EMBED_DIM)
  out3 = _gather(table, x.reshape(-1))
  return out3.reshape(BATCH, NUM_FIELDS * EMBED_DIM)
